# Initial kernel scaffold; baseline (speedup 1.0000x reference)
import jax
import jax.numpy as jnp
from jax.experimental import pallas as pl


def kernel(x, table, pe):
    raise NotImplementedError("write your pallas kernel here")


# speedup vs baseline: 4.2088x; 4.2088x over previous
import functools

import jax
import jax.numpy as jnp
from jax import lax
from jax.experimental import pallas as pl
from jax.experimental.pallas import tpu as pltpu
from jax.experimental.pallas import tpu_sc as plsc

_CHUNK = 128


@functools.lru_cache(maxsize=None)
def _make_sc_embed(N, T, V, D):
    info = plsc.get_sparse_core_info()
    NC, NS = info.num_cores, info.num_subcores
    NW = NC * NS
    assert N % (NW * _CHUNK) == 0
    rows_per_w = N // NW
    n_chunks = rows_per_w // _CHUNK
    assert rows_per_w % T == 0
    mesh = plsc.VectorSubcoreMesh(core_axis_name="c", subcore_axis_name="s")

    @functools.partial(
        pl.kernel,
        mesh=mesh,
        out_type=jax.ShapeDtypeStruct((N, D), jnp.float32),
        scratch_types=[
            pltpu.VMEM((_CHUNK,), jnp.int32),
            pltpu.VMEM((_CHUNK, D), jnp.float32),
            pltpu.VMEM_SHARED((2 * T, D), jnp.float32),
            pltpu.SemaphoreType.DMA,
        ],
    )
    def k(x_hbm, table_hbm, pe_hbm, out_hbm, idx_v, dest_v, pe2_sh, sem):
        wid = lax.axis_index("s") * NC + lax.axis_index("c")
        w_base = wid * rows_per_w
        @pl.when(lax.axis_index("s") == 0)
        def _init():
            pltpu.sync_copy(pe_hbm, pe2_sh.at[pl.ds(0, T)])
            pltpu.sync_copy(pe_hbm, pe2_sh.at[pl.ds(T, T)])

        plsc.subcore_barrier()

        def body(c, carry):
            base = w_base + c * _CHUNK
            off = lax.rem(c * _CHUNK, T)
            pltpu.sync_copy(x_hbm.at[pl.ds(base, _CHUNK)], idx_v)
            pltpu.sync_copy(pe2_sh.at[pl.ds(off, _CHUNK)], dest_v)
            pltpu.async_copy(table_hbm.at[idx_v], dest_v, sem, add=True).wait()
            pltpu.sync_copy(dest_v, out_hbm.at[pl.ds(base, _CHUNK)])
            return carry

        lax.fori_loop(0, n_chunks, body, 0)

    return k


def kernel(x, table, pe):
    B, T = x.shape
    V, D = table.shape
    f = _make_sc_embed(B * T, T, V, D)
    out = f(x.reshape(B * T).astype(jnp.int32), table, pe.reshape(T, D))
    return out.reshape(B, T, D)


# speedup vs baseline: 8.9531x; 2.1272x over previous
import functools

import jax
import jax.numpy as jnp
from jax import lax
from jax.experimental import pallas as pl
from jax.experimental.pallas import tpu as pltpu
from jax.experimental.pallas import tpu_sc as plsc

_CHUNK = 128
_NBUF = 4


@functools.lru_cache(maxsize=None)
def _make_sc_embed(N, T, V, D):
    info = plsc.get_sparse_core_info()
    NC, NS = info.num_cores, info.num_subcores
    NW = NC * NS
    assert N % (NW * _CHUNK) == 0
    rows_per_w = N // NW
    n_chunks = rows_per_w // _CHUNK
    assert n_chunks % _NBUF == 0
    n_groups = n_chunks // _NBUF
    assert rows_per_w % T == 0
    mesh = plsc.VectorSubcoreMesh(core_axis_name="c", subcore_axis_name="s")

    @functools.partial(
        pl.kernel,
        mesh=mesh,
        out_type=jax.ShapeDtypeStruct((N, D), jnp.float32),
        scratch_types=[
            pltpu.VMEM((n_chunks, _CHUNK), jnp.int32),
            pltpu.VMEM((_NBUF * _CHUNK, D), jnp.float32),
            pltpu.VMEM_SHARED((2 * T, D), jnp.float32),
        ]
        + [pltpu.SemaphoreType.DMA] * (2 * _NBUF),
    )
    def k(x2_hbm, table_hbm, pe_hbm, out_hbm, idx2d, dest, pe2_sh, *sems):
        gs, ws = sems[:_NBUF], sems[_NBUF:]
        wid = lax.axis_index("s") * NC + lax.axis_index("c")

        @pl.when(lax.axis_index("s") == 0)
        def _init_pe():
            pltpu.sync_copy(pe_hbm, pe2_sh.at[pl.ds(0, T)])
            pltpu.sync_copy(pe_hbm, pe2_sh.at[pl.ds(T, T)])

        plsc.subcore_barrier()
        pltpu.sync_copy(x2_hbm.at[pl.ds(wid * n_chunks, n_chunks)], idx2d)

        def dslice(b):
            return dest.at[pl.ds(b * _CHUNK, _CHUNK)]

        def out_slice(cid):
            return out_hbm.at[pl.ds((wid * n_chunks + cid) * _CHUNK, _CHUNK)]

        def group(g, carry):
            descs = {}
            for b in range(_NBUF):
                cid = g * _NBUF + b
                pb = (b - 1) % _NBUF

                @pl.when(g > 0)
                def _wb_done(b=b, cid=cid):
                    pltpu.make_async_copy(
                        dslice(b), out_slice(cid - _NBUF), ws[b]
                    ).wait()

                off = lax.rem(cid * _CHUNK, T)
                pltpu.sync_copy(pe2_sh.at[pl.ds(off, _CHUNK)], dslice(b))
                descs[b] = pltpu.async_copy(
                    table_hbm.at[idx2d.at[cid]], dslice(b), gs[b], add=True
                )

                if b > 0:
                    descs[b - 1].wait()
                    pltpu.async_copy(dslice(pb), out_slice(cid - 1), ws[pb])
                else:

                    @pl.when(g > 0)
                    def _prev(cid=cid, pb=pb):
                        pltpu.make_async_copy(
                            table_hbm.at[idx2d.at[cid - 1]], dslice(pb), gs[pb]
                        ).wait()
                        pltpu.async_copy(dslice(pb), out_slice(cid - 1), ws[pb])

            return carry

        lax.fori_loop(0, n_groups, group, 0)

        last = n_chunks - 1
        pltpu.make_async_copy(
            table_hbm.at[idx2d.at[last]], dslice(_NBUF - 1), gs[_NBUF - 1]
        ).wait()
        pltpu.async_copy(dslice(_NBUF - 1), out_slice(last), ws[_NBUF - 1])
        for b in range(_NBUF):
            pltpu.make_async_copy(
                dslice(b), out_slice(n_chunks - _NBUF + b), ws[b]
            ).wait()

    return k


def kernel(x, table, pe):
    B, T = x.shape
    V, D = table.shape
    f = _make_sc_embed(B * T, T, V, D)
    x2 = x.reshape(B * T // _CHUNK, _CHUNK).astype(jnp.int32)
    out = f(x2, table, pe.reshape(T, D))
    return out.reshape(B, T, D)


# speedup vs baseline: 9.1525x; 1.0223x over previous
import functools

import jax
import jax.numpy as jnp
from jax import lax
from jax.experimental import pallas as pl
from jax.experimental.pallas import tpu as pltpu
from jax.experimental.pallas import tpu_sc as plsc

_CHUNK = 128
_NBUF = 5


@functools.lru_cache(maxsize=None)
def _make_sc_embed(N, T, V, D):
    info = plsc.get_sparse_core_info()
    NC, NS = info.num_cores, info.num_subcores
    NW = NC * NS
    assert N % (NW * _CHUNK) == 0
    rows_per_w = N // NW
    n_chunks = rows_per_w // _CHUNK
    assert n_chunks % _NBUF == 0
    n_groups = n_chunks // _NBUF
    assert rows_per_w % T == 0
    mesh = plsc.VectorSubcoreMesh(core_axis_name="c", subcore_axis_name="s")

    @functools.partial(
        pl.kernel,
        mesh=mesh,
        out_type=jax.ShapeDtypeStruct((N, D), jnp.float32),
        scratch_types=[
            pltpu.VMEM((n_chunks, _CHUNK), jnp.int32),
            pltpu.VMEM((_NBUF * _CHUNK, D), jnp.float32),
            pltpu.VMEM_SHARED((2 * T, D), jnp.float32),
        ]
        + [pltpu.SemaphoreType.DMA] * (3 * _NBUF),
    )
    def k(x2_hbm, table_hbm, pe_hbm, out_hbm, idx2d, dest, pe2_sh, *sems):
        gs = sems[:_NBUF]
        ws = sems[_NBUF : 2 * _NBUF]
        isems = sems[2 * _NBUF :]
        wid = lax.axis_index("s") * NC + lax.axis_index("c")

        @pl.when(lax.axis_index("s") == 0)
        def _init_pe():
            pltpu.sync_copy(pe_hbm, pe2_sh.at[pl.ds(0, T)])
            pltpu.sync_copy(pe_hbm, pe2_sh.at[pl.ds(T, T)])

        plsc.subcore_barrier()
        pltpu.sync_copy(x2_hbm.at[pl.ds(wid * n_chunks, n_chunks)], idx2d)

        def dslice(b):
            return dest.at[pl.ds(b * _CHUNK, _CHUNK)]

        def out_slice(cid):
            return out_hbm.at[pl.ds((wid * n_chunks + cid) * _CHUNK, _CHUNK)]

        def fire_init(cid, b):
            off = lax.rem(cid * _CHUNK, T)
            pltpu.async_copy(pe2_sh.at[pl.ds(off, _CHUNK)], dslice(b), isems[b])

        def wait_init(b):
            pltpu.make_async_copy(
                pe2_sh.at[pl.ds(0, _CHUNK)], dslice(b), isems[b]
            ).wait()

        fire_init(0, 0)

        def group(g, carry):
            descs = {}
            for b in range(_NBUF):
                cid = g * _NBUF + b
                pb = (b - 1) % _NBUF
                nb = (b + 1) % _NBUF

                wait_init(b)
                descs[b] = pltpu.async_copy(
                    table_hbm.at[idx2d.at[cid]], dslice(b), gs[b], add=True
                )

                def _reclaim_and_init(cid=cid, nb=nb):
                    pltpu.make_async_copy(
                        dslice(nb), out_slice(cid + 1 - _NBUF), ws[nb]
                    ).wait()
                    fire_init(cid + 1, nb)

                if b == _NBUF - 1:
                    @pl.when(g + 1 < n_groups)
                    def _rc_last(cid=cid, nb=nb):
                        _reclaim_and_init(cid, nb)

                else:

                    @pl.when(g > 0)
                    def _rc(cid=cid, nb=nb):
                        _reclaim_and_init(cid, nb)

                    @pl.when(g == 0)
                    def _first(cid=cid, nb=nb):
                        fire_init(cid + 1, nb)

                if b > 0:
                    descs[b - 1].wait()
                    pltpu.async_copy(dslice(pb), out_slice(cid - 1), ws[pb])
                else:

                    @pl.when(g > 0)
                    def _prev(cid=cid, pb=pb):
                        pltpu.make_async_copy(
                            table_hbm.at[idx2d.at[cid - 1]], dslice(pb), gs[pb]
                        ).wait()
                        pltpu.async_copy(dslice(pb), out_slice(cid - 1), ws[pb])

            return carry

        lax.fori_loop(0, n_groups, group, 0)

        last = n_chunks - 1
        pltpu.make_async_copy(
            table_hbm.at[idx2d.at[last]], dslice(_NBUF - 1), gs[_NBUF - 1]
        ).wait()
        pltpu.async_copy(dslice(_NBUF - 1), out_slice(last), ws[_NBUF - 1])
        for b in range(_NBUF):
            pltpu.make_async_copy(
                dslice(b), out_slice(n_chunks - _NBUF + b), ws[b]
            ).wait()

    return k


def kernel(x, table, pe):
    B, T = x.shape
    V, D = table.shape
    f = _make_sc_embed(B * T, T, V, D)
    x2 = x.reshape(B * T // _CHUNK, _CHUNK).astype(jnp.int32)
    out = f(x2, table, pe.reshape(T, D))
    return out.reshape(B, T, D)
